# baseline (speedup 1.0000x reference)
import functools

import jax
import jax.numpy as jnp
from jax import lax
from jax.experimental import pallas as pl
from jax.experimental.pallas import tpu as pltpu
from jax.experimental.pallas import tpu_sc as plsc

B = 4
R = 8192
C = 4096
TOTAL = B * C
LANES = 16

_info = plsc.get_sparse_core_info()
NC = _info.num_cores
NS = _info.num_subcores
NW = NC * NS
PER_W = TOTAL // NW
WIN = 128
GRP = 64
NGROUP = PER_W // GRP

_mesh = plsc.VectorSubcoreMesh(core_axis_name="c", subcore_axis_name="s")


@functools.partial(
    pl.kernel,
    mesh=_mesh,
    out_type=jax.ShapeDtypeStruct((B, 1, C), jnp.float32),
    scratch_types=[
        pltpu.VMEM((PER_W,), jnp.int32),
        pltpu.VMEM((NGROUP, GRP, WIN), jnp.float32),
        pltpu.VMEM((PER_W,), jnp.float32),
    ]
    + [pltpu.SemaphoreType.DMA] * NGROUP,
    compiler_params=pltpu.CompilerParams(needs_layout_passes=False),
)
def _gather_kernel(x_hbm, idx_hbm, out_hbm, idx_v, vals_v, res_v, *sems):
    wid = lax.axis_index("s") * NC + lax.axis_index("c")
    base = wid * PER_W
    b = base // C
    c0 = base - b * C

    pltpu.sync_copy(idx_hbm.at[b, 0, pl.ds(c0, PER_W)], idx_v)

    for j in range(PER_W // LANES):
        sl = pl.ds(j * LANES, LANES)
        idx_v[sl] = idx_v[sl] + (b * R)

    copies = []
    for g in range(NGROUP):
        rows = idx_v.at[pl.ds(g * GRP, GRP)]
        cw = c0 + (g // 2) * WIN
        copies.append(
            pltpu.async_copy(
                x_hbm.at[rows, pl.ds(cw, WIN)], vals_v.at[g], sems[g]
            )
        )

    lane = lax.iota(jnp.int32, LANES)
    for g in range(NGROUP):
        copies[g].wait()
        for j in range(GRP // LANES):
            rowsel = lane + (j * LANES)
            colsel = rowsel + ((g % 2) * GRP)
            res_v[pl.ds(g * GRP + j * LANES, LANES)] = plsc.load_gather(
                vals_v.at[g], [rowsel, colsel]
            )

    pltpu.sync_copy(res_v, out_hbm.at[b, 0, pl.ds(c0, PER_W)])


def kernel(x, index):
    x2d = x.reshape(B * R, C)
    return _gather_kernel(x2d, index)
